# baseline (speedup 1.0000x reference)
import functools

import jax
import jax.numpy as jnp
from jax.experimental import pallas as pl
from jax.experimental.pallas import tpu as pltpu

_EPS = 1e-5


def _rup(v, m):
    return (v + m - 1) // m * m


def _fused_mm_bn_body(*refs, relu, has_res, inv_m, tm):
    if has_res:
        a_ref, w_ref, g_ref, b_ref, r_ref, o_ref, y_scr, s_scr, ss_scr = refs
    else:
        a_ref, w_ref, g_ref, b_ref, o_ref, y_scr, s_scr, ss_scr = refs
        r_ref = None
    ph = pl.program_id(0)
    it = pl.program_id(1)

    @pl.when(ph == 0)
    def _matmul_phase():
        acc = jnp.dot(a_ref[...], w_ref[...], preferred_element_type=jnp.float32)
        y_scr[pl.ds(it * tm, tm), :] = acc
        cs = jnp.sum(acc, axis=0, keepdims=True)
        css = jnp.sum(acc * acc, axis=0, keepdims=True)

        @pl.when(it == 0)
        def _init():
            s_scr[...] = cs
            ss_scr[...] = css

        @pl.when(it != 0)
        def _accum():
            s_scr[...] = s_scr[...] + cs
            ss_scr[...] = ss_scr[...] + css

    @pl.when(ph == 1)
    def _normalize_phase():
        mean = s_scr[...] * inv_m
        var = jnp.maximum(ss_scr[...] * inv_m - mean * mean, 0.0)
        scale = g_ref[...] * jax.lax.rsqrt(var + _EPS)
        shift = b_ref[...] - mean * scale
        y = y_scr[pl.ds(it * tm, tm), :] * scale + shift
        if has_res:
            y = y + r_ref[...].astype(jnp.float32)
        if relu:
            y = jnp.maximum(y, 0.0)
        o_ref[...] = y.astype(o_ref.dtype)


def _mm_bn(a, w, gamma, beta, residual=None, relu=True):
    M, K = a.shape
    Nc = w.shape[1]
    Kp, Np = K, Nc

    tm = min(_rup(M, 16), 2048)
    while tm > 256 and (4 * tm * Kp + _rup(M, tm) * Np * 4
                        + 2 * Kp * Np) > 20 * 1024 * 1024:
        tm //= 2
    Mp = _rup(M, tm)
    nt = Mp // tm

    a_p = jnp.pad(a.astype(jnp.bfloat16), ((0, Mp - M), (0, Kp - K)))
    w_p = jnp.pad(w.astype(jnp.bfloat16), ((0, Kp - K), (0, Np - Nc)))
    g_p = jnp.pad(gamma.astype(jnp.float32), (0, Np - Nc)).reshape(1, Np)
    b_p = jnp.pad(beta.astype(jnp.float32), (0, Np - Nc)).reshape(1, Np)

    args = [a_p, w_p, g_p, b_p]
    in_specs = [
        pl.BlockSpec((tm, Kp), lambda p, i: (i * (1 - p), 0)),
        pl.BlockSpec((Kp, Np), lambda p, i: (0, 0)),
        pl.BlockSpec((1, Np), lambda p, i: (0, 0)),
        pl.BlockSpec((1, Np), lambda p, i: (0, 0)),
    ]
    if residual is not None:
        r_p = jnp.pad(residual.astype(jnp.bfloat16),
                      ((0, Mp - M), (0, Np - Nc)))
        args.append(r_p)
        in_specs.append(pl.BlockSpec((tm, Np), lambda p, i: (i * p, 0)))

    out = pl.pallas_call(
        functools.partial(_fused_mm_bn_body, relu=relu,
                          has_res=residual is not None,
                          inv_m=1.0 / float(M), tm=tm),
        out_shape=jax.ShapeDtypeStruct((Mp, Np), jnp.bfloat16),
        grid=(2, nt),
        in_specs=in_specs,
        out_specs=pl.BlockSpec((tm, Np), lambda p, i: (i * p, 0)),
        scratch_shapes=[pltpu.VMEM((Mp, Np), jnp.float32),
                        pltpu.VMEM((1, Np), jnp.float32),
                        pltpu.VMEM((1, Np), jnp.float32)],
        compiler_params=pltpu.CompilerParams(
            dimension_semantics=("arbitrary", "arbitrary")),
    )(*args)
    return out[:M, :Nc]


def _patches(x, ksize, stride, padding):
    N, D, H, W, C = x.shape
    kd, kh, kw = ksize
    sd, sh, sw = stride
    pd, ph, pw = padding
    Do = (D + 2 * pd - kd) // sd + 1
    Ho = (H + 2 * ph - kh) // sh + 1
    Wo = (W + 2 * pw - kw) // sw + 1
    if ksize == (1, 1, 1):
        sl = x[:, ::sd, ::sh, ::sw, :]
        return sl.reshape(N * Do * Ho * Wo, C), (Do, Ho, Wo)
    dn = ("NDHWC", "DHWIO", "NDHWC")
    A = x.astype(jnp.bfloat16)
    A = jax.lax.conv_general_dilated_patches(
        A, (kd, 1, 1), (sd, 1, 1), [(pd, pd), (0, 0), (0, 0)],
        dimension_numbers=dn)
    A = jax.lax.conv_general_dilated_patches(
        A, (1, kh, 1), (1, sh, 1), [(0, 0), (ph, ph), (0, 0)],
        dimension_numbers=dn)
    A = jax.lax.conv_general_dilated_patches(
        A, (1, 1, kw), (1, 1, sw), [(0, 0), (0, 0), (pw, pw)],
        dimension_numbers=dn)
    return A.reshape(N * Do * Ho * Wo, kd * kh * kw * C), (Do, Ho, Wo)


def _conv(x, w, g, b, ksize, stride, padding, relu=True, residual=None):
    A, (Do, Ho, Wo) = _patches(x, ksize, stride, padding)
    N = x.shape[0]
    Nc = w.shape[1]
    res = residual.reshape(-1, Nc) if residual is not None else None
    y = _mm_bn(A, w, g, b, residual=res, relu=relu)
    return y.reshape(N, Do, Ho, Wo, Nc)


def _stem_body(bp_ref, w_ref, g_ref, b_ref, o_ref):
    bp = bp_ref[...].reshape(8, 11, 2, 16, 16, 49)
    wv = w_ref[...].astype(jnp.bfloat16)
    M = 8 * 8 * 16 * 16
    acc = jnp.zeros((M, 32), jnp.float32)
    for i in range(7):
        p, a = i % 2, i // 2
        u = bp[:, a:a + 8, p].reshape(M, 49)
        acc = acc + jnp.dot(u, wv[i * 49:(i + 1) * 49, :],
                            preferred_element_type=jnp.float32)
    sc, sh = _bn_from_acc(acc, g_ref, b_ref, M)
    y = jnp.maximum(acc * sc + sh, 0.0).astype(jnp.bfloat16)
    o_ref[...] = y.reshape(8, 8, 16, 16, 32)


def _stem(x5, w, g, b):
    hw = jax.lax.conv_general_dilated_patches(
        x5, (1, 7, 7), (1, 2, 2), [(0, 0), (3, 3), (3, 3)],
        dimension_numbers=("NDHWC", "DHWIO", "NDHWC"))
    bp = jnp.pad(hw, ((0, 0), (3, 3), (0, 0), (0, 0), (0, 0)))
    return pl.pallas_call(
        _stem_body,
        out_shape=jax.ShapeDtypeStruct((8, 8, 16, 16, 32), jnp.bfloat16),
    )(bp, w, g.reshape(1, 32), b.reshape(1, 32))


def _bn_from_acc(acc, g_ref, b_ref, m_rows):
    inv_m = 1.0 / float(m_rows)
    mean = jnp.sum(acc, axis=0, keepdims=True) * inv_m
    ex2 = jnp.sum(acc * acc, axis=0, keepdims=True) * inv_m
    var = jnp.maximum(ex2 - mean * mean, 0.0)
    scale = g_ref[...] * jax.lax.rsqrt(var + _EPS)
    shift = b_ref[...] - mean * scale
    return scale, shift


def _conv3_s1_acc(xv, wv):
    N, D, H, W, C = xv.shape
    C2 = wv.shape[1]
    M = N * D * H * W
    xp = jnp.pad(xv, ((0, 0), (1, 1), (1, 1), (1, 1), (0, 0)))
    acc = jnp.zeros((M, C2), jnp.float32)
    for i in range(3):
        pieces = [xp[:, i:i + D, j:j + H, k:k + W, :]
                  for j in range(3) for k in range(3)]
        u = jnp.concatenate(pieces, axis=-1).reshape(M, 9 * C)
        acc = acc + jnp.dot(u, wv[i * 9 * C:(i + 1) * 9 * C, :],
                            preferred_element_type=jnp.float32)
    return acc


def _conv3_s2_acc(phases, wv, out_sp):
    Do, Ho, Wo = out_sp
    C = phases[0][0][0].shape[4]
    N = phases[0][0][0].shape[0]
    C2 = wv.shape[1]
    M = N * Do * Ho * Wo
    acc = jnp.zeros((M, C2), jnp.float32)
    for i in range(3):
        pieces = []
        for j in range(3):
            for k in range(3):
                ph = phases[i % 2][j % 2][k % 2]
                oi, oj, ok = i // 2, j // 2, k // 2
                pieces.append(ph[:, oi:oi + Do, oj:oj + Ho, ok:ok + Wo, :])
        u = jnp.concatenate(pieces, axis=-1).reshape(M, 9 * C)
        acc = acc + jnp.dot(u, wv[i * 9 * C:(i + 1) * 9 * C, :],
                            preferred_element_type=jnp.float32)
    return acc


def _block_s1_body(x_ref, w1_ref, g1_ref, b1_ref, w2_ref, g2_ref, b2_ref,
                   o_ref):
    xv = x_ref[...]
    N, D, H, W, C = xv.shape
    M = N * D * H * W
    C2 = w1_ref.shape[1]

    acc1 = _conv3_s1_acc(xv, w1_ref[...].astype(jnp.bfloat16))
    sc, sh = _bn_from_acc(acc1, g1_ref, b1_ref, M)
    h1 = jnp.maximum(acc1 * sc + sh, 0.0).astype(jnp.bfloat16)

    acc2 = _conv3_s1_acc(h1.reshape(N, D, H, W, C2),
                         w2_ref[...].astype(jnp.bfloat16))
    sc2, sh2 = _bn_from_acc(acc2, g2_ref, b2_ref, M)
    y = acc2 * sc2 + sh2 + xv.reshape(M, C).astype(jnp.float32)
    y = jnp.maximum(y, 0.0).astype(jnp.bfloat16)
    o_ref[...] = y.reshape(N, D, H, W, C2)


def _block_s2_body(*refs, out_sp):
    (x_ref,
     w1_ref, g1_ref, b1_ref, w2_ref, g2_ref, b2_ref,
     wd_ref, gd_ref, bd_ref, o_ref) = refs
    xp = jnp.pad(x_ref[...], ((0, 0), (1, 1), (1, 1), (1, 1), (0, 0)))
    N, Dp, Hp, Wp, C = xp.shape
    xr = xp.reshape(N, Dp // 2, 2, Hp // 2, 2, Wp // 2, 2, C)
    phases = [[[xr[:, :, p, :, q, :, r, :] for r in (0, 1)]
               for q in (0, 1)] for p in (0, 1)]
    Do, Ho, Wo = out_sp
    M = N * Do * Ho * Wo
    C2 = w1_ref.shape[1]

    acc1 = _conv3_s2_acc(phases, w1_ref[...].astype(jnp.bfloat16), out_sp)
    sc, sh = _bn_from_acc(acc1, g1_ref, b1_ref, M)
    h1 = jnp.maximum(acc1 * sc + sh, 0.0).astype(jnp.bfloat16)

    rs = phases[1][1][1][:, 0:Do, 0:Ho, 0:Wo, :].reshape(M, C)
    accd = jnp.dot(rs, wd_ref[...].astype(jnp.bfloat16),
                   preferred_element_type=jnp.float32)
    scd, shd = _bn_from_acc(accd, gd_ref, bd_ref, M)
    res = accd * scd + shd

    acc2 = _conv3_s1_acc(h1.reshape(N, Do, Ho, Wo, C2),
                         w2_ref[...].astype(jnp.bfloat16))
    sc2, sh2 = _bn_from_acc(acc2, g2_ref, b2_ref, M)
    y = acc2 * sc2 + sh2 + res.astype(jnp.bfloat16).astype(jnp.float32)
    y = jnp.maximum(y, 0.0).astype(jnp.bfloat16)
    o_ref[...] = y.reshape(N, Do, Ho, Wo, C2)


def _basic_block(h, c1, c2, ds, stride):
    N, D, H, W, C = h.shape
    C2 = c1[0].shape[1]

    def prep(cw):
        w, g, b = cw
        return [w, g.reshape(1, C2), b.reshape(1, C2)]

    h = h.astype(jnp.bfloat16)
    if stride == 1:
        args = [h] + prep(c1) + prep(c2)
        return pl.pallas_call(
            _block_s1_body,
            out_shape=jax.ShapeDtypeStruct((N, D, H, W, C2), jnp.bfloat16),
        )(*args)

    Do, Ho, Wo = D // 2, H // 2, W // 2
    args = [h] + prep(c1) + prep(c2) + prep(ds)
    return pl.pallas_call(
        functools.partial(_block_s2_body, out_sp=(Do, Ho, Wo)),
        out_shape=jax.ShapeDtypeStruct((N, Do, Ho, Wo, C2), jnp.bfloat16),
    )(*args)


def _head_body(x_ref, w0_ref, b0_ref, w1_ref, b1_ref, w2_ref, b2_ref, o_ref):
    e = jnp.dot(x_ref[...], w0_ref[...],
                preferred_element_type=jnp.float32) + b0_ref[...]
    e = jnp.dot(e, w1_ref[...], preferred_element_type=jnp.float32) + b1_ref[...]
    o_ref[...] = jnp.dot(e, w2_ref[...],
                         preferred_element_type=jnp.float32) + b2_ref[...]


def _finetune_head(pool, con0_w, con0_b, con1_w, con1_b, ft_w, ft_b):
    M = pool.shape[0]
    args = [pool, con0_w, con0_b.reshape(1, -1), con1_w,
            con1_b.reshape(1, -1), ft_w, ft_b.reshape(1, -1)]
    return pl.pallas_call(
        _head_body,
        out_shape=jax.ShapeDtypeStruct((M, 1), jnp.float32),
    )(*args)


def kernel(x, stem_w, stem_gamma, stem_beta, l1b0_c1_w, l1b0_c1_g, l1b0_c1_b, l1b0_c2_w, l1b0_c2_g, l1b0_c2_b, l1b1_c1_w, l1b1_c1_g, l1b1_c1_b, l1b1_c2_w, l1b1_c2_g, l1b1_c2_b, l2b0_c1_w, l2b0_c1_g, l2b0_c1_b, l2b0_c2_w, l2b0_c2_g, l2b0_c2_b, l2b0_ds_w, l2b0_ds_g, l2b0_ds_b, l2b1_c1_w, l2b1_c1_g, l2b1_c1_b, l2b1_c2_w, l2b1_c2_g, l2b1_c2_b, l3b0_c1_w, l3b0_c1_g, l3b0_c1_b, l3b0_c2_w, l3b0_c2_g, l3b0_c2_b, l3b0_ds_w, l3b0_ds_g, l3b0_ds_b, l3b1_c1_w, l3b1_c1_g, l3b1_c1_b, l3b1_c2_w, l3b1_c2_g, l3b1_c2_b, l4b0_c1_w, l4b0_c1_g, l4b0_c1_b, l4b0_c2_w, l4b0_c2_g, l4b0_c2_b, l4b0_ds_w, l4b0_ds_g, l4b0_ds_b, l4b1_c1_w, l4b1_c1_g, l4b1_c1_b, l4b1_c2_w, l4b1_c2_g, l4b1_c2_b, fc_w, fc_g, fc_b, cls0_w, cls0_b, cls1_w, cls1_b, cls2_w, cls2_b, con0_w, con0_b, con1_w, con1_b, ft_w, ft_b):
    h = x.reshape(x.shape[0], *x.shape[2:], 1).astype(jnp.bfloat16)
    h = _stem(h, stem_w, stem_gamma, stem_beta)

    h = _basic_block(h, (l1b0_c1_w, l1b0_c1_g, l1b0_c1_b),
                     (l1b0_c2_w, l1b0_c2_g, l1b0_c2_b), None, 1)
    h = _basic_block(h, (l1b1_c1_w, l1b1_c1_g, l1b1_c1_b),
                     (l1b1_c2_w, l1b1_c2_g, l1b1_c2_b), None, 1)

    h = _basic_block(h, (l2b0_c1_w, l2b0_c1_g, l2b0_c1_b),
                     (l2b0_c2_w, l2b0_c2_g, l2b0_c2_b),
                     (l2b0_ds_w, l2b0_ds_g, l2b0_ds_b), 2)
    h = _basic_block(h, (l2b1_c1_w, l2b1_c1_g, l2b1_c1_b),
                     (l2b1_c2_w, l2b1_c2_g, l2b1_c2_b), None, 1)

    h = _basic_block(h, (l3b0_c1_w, l3b0_c1_g, l3b0_c1_b),
                     (l3b0_c2_w, l3b0_c2_g, l3b0_c2_b),
                     (l3b0_ds_w, l3b0_ds_g, l3b0_ds_b), 2)
    h = _basic_block(h, (l3b1_c1_w, l3b1_c1_g, l3b1_c1_b),
                     (l3b1_c2_w, l3b1_c2_g, l3b1_c2_b), None, 1)

    h = _basic_block(h, (l4b0_c1_w, l4b0_c1_g, l4b0_c1_b),
                     (l4b0_c2_w, l4b0_c2_g, l4b0_c2_b),
                     (l4b0_ds_w, l4b0_ds_g, l4b0_ds_b), 2)
    h = _basic_block(h, (l4b1_c1_w, l4b1_c1_g, l4b1_c1_b),
                     (l4b1_c2_w, l4b1_c2_g, l4b1_c2_b), None, 1)

    h = _conv(h, fc_w, fc_g, fc_b, (1, 1, 1), (1, 1, 1), (0, 0, 0), relu=True)
    pool = jnp.mean(h.astype(jnp.float32), axis=(1, 2, 3))
    return _finetune_head(pool, con0_w, con0_b, con1_w, con1_b, ft_w, ft_b)


# speedup vs baseline: 1.1320x; 1.1320x over previous
import functools

import jax
import jax.numpy as jnp
from jax.experimental import pallas as pl
from jax.experimental.pallas import tpu as pltpu

_EPS = 1e-5


def _rup(v, m):
    return (v + m - 1) // m * m


def _fused_mm_bn_body(*refs, relu, has_res, inv_m, tm):
    if has_res:
        a_ref, w_ref, g_ref, b_ref, r_ref, o_ref, y_scr, s_scr, ss_scr = refs
    else:
        a_ref, w_ref, g_ref, b_ref, o_ref, y_scr, s_scr, ss_scr = refs
        r_ref = None
    ph = pl.program_id(0)
    it = pl.program_id(1)

    @pl.when(ph == 0)
    def _matmul_phase():
        acc = jnp.dot(a_ref[...], w_ref[...], preferred_element_type=jnp.float32)
        y_scr[pl.ds(it * tm, tm), :] = acc
        cs = jnp.sum(acc, axis=0, keepdims=True)
        css = jnp.sum(acc * acc, axis=0, keepdims=True)

        @pl.when(it == 0)
        def _init():
            s_scr[...] = cs
            ss_scr[...] = css

        @pl.when(it != 0)
        def _accum():
            s_scr[...] = s_scr[...] + cs
            ss_scr[...] = ss_scr[...] + css

    @pl.when(ph == 1)
    def _normalize_phase():
        mean = s_scr[...] * inv_m
        var = jnp.maximum(ss_scr[...] * inv_m - mean * mean, 0.0)
        scale = g_ref[...] * jax.lax.rsqrt(var + _EPS)
        shift = b_ref[...] - mean * scale
        y = y_scr[pl.ds(it * tm, tm), :] * scale + shift
        if has_res:
            y = y + r_ref[...].astype(jnp.float32)
        if relu:
            y = jnp.maximum(y, 0.0)
        o_ref[...] = y.astype(o_ref.dtype)


def _mm_bn(a, w, gamma, beta, residual=None, relu=True):
    M, K = a.shape
    Nc = w.shape[1]
    Kp, Np = K, Nc

    tm = min(_rup(M, 16), 2048)
    while tm > 256 and (4 * tm * Kp + _rup(M, tm) * Np * 4
                        + 2 * Kp * Np) > 20 * 1024 * 1024:
        tm //= 2
    Mp = _rup(M, tm)
    nt = Mp // tm

    a_p = jnp.pad(a.astype(jnp.bfloat16), ((0, Mp - M), (0, Kp - K)))
    w_p = jnp.pad(w.astype(jnp.bfloat16), ((0, Kp - K), (0, Np - Nc)))
    g_p = jnp.pad(gamma.astype(jnp.float32), (0, Np - Nc)).reshape(1, Np)
    b_p = jnp.pad(beta.astype(jnp.float32), (0, Np - Nc)).reshape(1, Np)

    args = [a_p, w_p, g_p, b_p]
    in_specs = [
        pl.BlockSpec((tm, Kp), lambda p, i: (i * (1 - p), 0)),
        pl.BlockSpec((Kp, Np), lambda p, i: (0, 0)),
        pl.BlockSpec((1, Np), lambda p, i: (0, 0)),
        pl.BlockSpec((1, Np), lambda p, i: (0, 0)),
    ]
    if residual is not None:
        r_p = jnp.pad(residual.astype(jnp.bfloat16),
                      ((0, Mp - M), (0, Np - Nc)))
        args.append(r_p)
        in_specs.append(pl.BlockSpec((tm, Np), lambda p, i: (i * p, 0)))

    out = pl.pallas_call(
        functools.partial(_fused_mm_bn_body, relu=relu,
                          has_res=residual is not None,
                          inv_m=1.0 / float(M), tm=tm),
        out_shape=jax.ShapeDtypeStruct((Mp, Np), jnp.bfloat16),
        grid=(2, nt),
        in_specs=in_specs,
        out_specs=pl.BlockSpec((tm, Np), lambda p, i: (i * p, 0)),
        scratch_shapes=[pltpu.VMEM((Mp, Np), jnp.float32),
                        pltpu.VMEM((1, Np), jnp.float32),
                        pltpu.VMEM((1, Np), jnp.float32)],
        compiler_params=pltpu.CompilerParams(
            dimension_semantics=("arbitrary", "arbitrary")),
    )(*args)
    return out[:M, :Nc]


def _patches(x, ksize, stride, padding):
    N, D, H, W, C = x.shape
    kd, kh, kw = ksize
    sd, sh, sw = stride
    pd, ph, pw = padding
    Do = (D + 2 * pd - kd) // sd + 1
    Ho = (H + 2 * ph - kh) // sh + 1
    Wo = (W + 2 * pw - kw) // sw + 1
    if ksize == (1, 1, 1):
        sl = x[:, ::sd, ::sh, ::sw, :]
        return sl.reshape(N * Do * Ho * Wo, C), (Do, Ho, Wo)
    dn = ("NDHWC", "DHWIO", "NDHWC")
    A = x.astype(jnp.bfloat16)
    A = jax.lax.conv_general_dilated_patches(
        A, (kd, 1, 1), (sd, 1, 1), [(pd, pd), (0, 0), (0, 0)],
        dimension_numbers=dn)
    A = jax.lax.conv_general_dilated_patches(
        A, (1, kh, 1), (1, sh, 1), [(0, 0), (ph, ph), (0, 0)],
        dimension_numbers=dn)
    A = jax.lax.conv_general_dilated_patches(
        A, (1, 1, kw), (1, 1, sw), [(0, 0), (0, 0), (pw, pw)],
        dimension_numbers=dn)
    return A.reshape(N * Do * Ho * Wo, kd * kh * kw * C), (Do, Ho, Wo)


def _conv(x, w, g, b, ksize, stride, padding, relu=True, residual=None):
    A, (Do, Ho, Wo) = _patches(x, ksize, stride, padding)
    N = x.shape[0]
    Nc = w.shape[1]
    res = residual.reshape(-1, Nc) if residual is not None else None
    y = _mm_bn(A, w, g, b, residual=res, relu=relu)
    return y.reshape(N, Do, Ho, Wo, Nc)


def _bn_from_acc(acc, g_ref, b_ref, m_rows):
    inv_m = 1.0 / float(m_rows)
    mean = jnp.sum(acc, axis=0, keepdims=True) * inv_m
    ex2 = jnp.sum(acc * acc, axis=0, keepdims=True) * inv_m
    var = jnp.maximum(ex2 - mean * mean, 0.0)
    scale = g_ref[...] * jax.lax.rsqrt(var + _EPS)
    shift = b_ref[...] - mean * scale
    return scale, shift


def _conv3_s1_acc(xv, wv):
    N, D, H, W, C = xv.shape
    C2 = wv.shape[1]
    M = N * D * H * W
    xp = jnp.pad(xv, ((0, 0), (1, 1), (1, 1), (1, 1), (0, 0)))
    acc = jnp.zeros((M, C2), jnp.float32)
    for i in range(3):
        pieces = [xp[:, i:i + D, j:j + H, k:k + W, :]
                  for j in range(3) for k in range(3)]
        u = jnp.concatenate(pieces, axis=-1).reshape(M, 9 * C)
        acc = acc + jnp.dot(u, wv[i * 9 * C:(i + 1) * 9 * C, :],
                            preferred_element_type=jnp.float32)
    return acc


def _conv3_s2_acc(phases, wv, out_sp):
    Do, Ho, Wo = out_sp
    C = phases[0][0][0].shape[4]
    N = phases[0][0][0].shape[0]
    C2 = wv.shape[1]
    M = N * Do * Ho * Wo
    acc = jnp.zeros((M, C2), jnp.float32)
    for i in range(3):
        pieces = []
        for j in range(3):
            for k in range(3):
                ph = phases[i % 2][j % 2][k % 2]
                oi, oj, ok = i // 2, j // 2, k // 2
                pieces.append(ph[:, oi:oi + Do, oj:oj + Ho, ok:ok + Wo, :])
        u = jnp.concatenate(pieces, axis=-1).reshape(M, 9 * C)
        acc = acc + jnp.dot(u, wv[i * 9 * C:(i + 1) * 9 * C, :],
                            preferred_element_type=jnp.float32)
    return acc


def _block_s1_body(x_ref, w1_ref, g1_ref, b1_ref, w2_ref, g2_ref, b2_ref,
                   o_ref):
    xv = x_ref[...]
    N, D, H, W, C = xv.shape
    M = N * D * H * W
    C2 = w1_ref.shape[1]

    acc1 = _conv3_s1_acc(xv, w1_ref[...].astype(jnp.bfloat16))
    sc, sh = _bn_from_acc(acc1, g1_ref, b1_ref, M)
    h1 = jnp.maximum(acc1 * sc + sh, 0.0).astype(jnp.bfloat16)

    acc2 = _conv3_s1_acc(h1.reshape(N, D, H, W, C2),
                         w2_ref[...].astype(jnp.bfloat16))
    sc2, sh2 = _bn_from_acc(acc2, g2_ref, b2_ref, M)
    y = acc2 * sc2 + sh2 + xv.reshape(M, C).astype(jnp.float32)
    y = jnp.maximum(y, 0.0).astype(jnp.bfloat16)
    o_ref[...] = y.reshape(N, D, H, W, C2)


def _block_s2_body(*refs, out_sp):
    (x_ref,
     w1_ref, g1_ref, b1_ref, w2_ref, g2_ref, b2_ref,
     wd_ref, gd_ref, bd_ref, o_ref) = refs
    xp = jnp.pad(x_ref[...], ((0, 0), (1, 1), (1, 1), (1, 1), (0, 0)))
    N, Dp, Hp, Wp, C = xp.shape
    xr = xp.reshape(N, Dp // 2, 2, Hp // 2, 2, Wp // 2, 2, C)
    phases = [[[xr[:, :, p, :, q, :, r, :] for r in (0, 1)]
               for q in (0, 1)] for p in (0, 1)]
    Do, Ho, Wo = out_sp
    M = N * Do * Ho * Wo
    C2 = w1_ref.shape[1]

    acc1 = _conv3_s2_acc(phases, w1_ref[...].astype(jnp.bfloat16), out_sp)
    sc, sh = _bn_from_acc(acc1, g1_ref, b1_ref, M)
    h1 = jnp.maximum(acc1 * sc + sh, 0.0).astype(jnp.bfloat16)

    rs = phases[1][1][1][:, 0:Do, 0:Ho, 0:Wo, :].reshape(M, C)
    accd = jnp.dot(rs, wd_ref[...].astype(jnp.bfloat16),
                   preferred_element_type=jnp.float32)
    scd, shd = _bn_from_acc(accd, gd_ref, bd_ref, M)
    res = accd * scd + shd

    acc2 = _conv3_s1_acc(h1.reshape(N, Do, Ho, Wo, C2),
                         w2_ref[...].astype(jnp.bfloat16))
    sc2, sh2 = _bn_from_acc(acc2, g2_ref, b2_ref, M)
    y = acc2 * sc2 + sh2 + res.astype(jnp.bfloat16).astype(jnp.float32)
    y = jnp.maximum(y, 0.0).astype(jnp.bfloat16)
    o_ref[...] = y.reshape(N, Do, Ho, Wo, C2)


def _basic_block(h, c1, c2, ds, stride):
    N, D, H, W, C = h.shape
    C2 = c1[0].shape[1]

    def prep(cw):
        w, g, b = cw
        return [w, g.reshape(1, C2), b.reshape(1, C2)]

    h = h.astype(jnp.bfloat16)
    if stride == 1:
        args = [h] + prep(c1) + prep(c2)
        return pl.pallas_call(
            _block_s1_body,
            out_shape=jax.ShapeDtypeStruct((N, D, H, W, C2), jnp.bfloat16),
        )(*args)

    Do, Ho, Wo = D // 2, H // 2, W // 2
    args = [h] + prep(c1) + prep(c2) + prep(ds)
    return pl.pallas_call(
        functools.partial(_block_s2_body, out_sp=(Do, Ho, Wo)),
        out_shape=jax.ShapeDtypeStruct((N, Do, Ho, Wo, C2), jnp.bfloat16),
    )(*args)


def _head_body(x_ref, w0_ref, b0_ref, w1_ref, b1_ref, w2_ref, b2_ref, o_ref):
    e = jnp.dot(x_ref[...], w0_ref[...],
                preferred_element_type=jnp.float32) + b0_ref[...]
    e = jnp.dot(e, w1_ref[...], preferred_element_type=jnp.float32) + b1_ref[...]
    o_ref[...] = jnp.dot(e, w2_ref[...],
                         preferred_element_type=jnp.float32) + b2_ref[...]


def _finetune_head(pool, con0_w, con0_b, con1_w, con1_b, ft_w, ft_b):
    M = pool.shape[0]
    args = [pool, con0_w, con0_b.reshape(1, -1), con1_w,
            con1_b.reshape(1, -1), ft_w, ft_b.reshape(1, -1)]
    return pl.pallas_call(
        _head_body,
        out_shape=jax.ShapeDtypeStruct((M, 1), jnp.float32),
    )(*args)


def kernel(x, stem_w, stem_gamma, stem_beta, l1b0_c1_w, l1b0_c1_g, l1b0_c1_b, l1b0_c2_w, l1b0_c2_g, l1b0_c2_b, l1b1_c1_w, l1b1_c1_g, l1b1_c1_b, l1b1_c2_w, l1b1_c2_g, l1b1_c2_b, l2b0_c1_w, l2b0_c1_g, l2b0_c1_b, l2b0_c2_w, l2b0_c2_g, l2b0_c2_b, l2b0_ds_w, l2b0_ds_g, l2b0_ds_b, l2b1_c1_w, l2b1_c1_g, l2b1_c1_b, l2b1_c2_w, l2b1_c2_g, l2b1_c2_b, l3b0_c1_w, l3b0_c1_g, l3b0_c1_b, l3b0_c2_w, l3b0_c2_g, l3b0_c2_b, l3b0_ds_w, l3b0_ds_g, l3b0_ds_b, l3b1_c1_w, l3b1_c1_g, l3b1_c1_b, l3b1_c2_w, l3b1_c2_g, l3b1_c2_b, l4b0_c1_w, l4b0_c1_g, l4b0_c1_b, l4b0_c2_w, l4b0_c2_g, l4b0_c2_b, l4b0_ds_w, l4b0_ds_g, l4b0_ds_b, l4b1_c1_w, l4b1_c1_g, l4b1_c1_b, l4b1_c2_w, l4b1_c2_g, l4b1_c2_b, fc_w, fc_g, fc_b, cls0_w, cls0_b, cls1_w, cls1_b, cls2_w, cls2_b, con0_w, con0_b, con1_w, con1_b, ft_w, ft_b):
    h = x.reshape(x.shape[0], *x.shape[2:], 1).astype(jnp.bfloat16)
    h = _conv(h, stem_w, stem_gamma, stem_beta,
              (7, 7, 7), (2, 2, 2), (3, 3, 3), relu=True)

    h = _basic_block(h, (l1b0_c1_w, l1b0_c1_g, l1b0_c1_b),
                     (l1b0_c2_w, l1b0_c2_g, l1b0_c2_b), None, 1)
    h = _basic_block(h, (l1b1_c1_w, l1b1_c1_g, l1b1_c1_b),
                     (l1b1_c2_w, l1b1_c2_g, l1b1_c2_b), None, 1)

    h = _basic_block(h, (l2b0_c1_w, l2b0_c1_g, l2b0_c1_b),
                     (l2b0_c2_w, l2b0_c2_g, l2b0_c2_b),
                     (l2b0_ds_w, l2b0_ds_g, l2b0_ds_b), 2)
    h = _basic_block(h, (l2b1_c1_w, l2b1_c1_g, l2b1_c1_b),
                     (l2b1_c2_w, l2b1_c2_g, l2b1_c2_b), None, 1)

    h = _basic_block(h, (l3b0_c1_w, l3b0_c1_g, l3b0_c1_b),
                     (l3b0_c2_w, l3b0_c2_g, l3b0_c2_b),
                     (l3b0_ds_w, l3b0_ds_g, l3b0_ds_b), 2)
    h = _basic_block(h, (l3b1_c1_w, l3b1_c1_g, l3b1_c1_b),
                     (l3b1_c2_w, l3b1_c2_g, l3b1_c2_b), None, 1)

    h = _basic_block(h, (l4b0_c1_w, l4b0_c1_g, l4b0_c1_b),
                     (l4b0_c2_w, l4b0_c2_g, l4b0_c2_b),
                     (l4b0_ds_w, l4b0_ds_g, l4b0_ds_b), 2)
    h = _basic_block(h, (l4b1_c1_w, l4b1_c1_g, l4b1_c1_b),
                     (l4b1_c2_w, l4b1_c2_g, l4b1_c2_b), None, 1)

    h = _conv(h, fc_w, fc_g, fc_b, (1, 1, 1), (1, 1, 1), (0, 0, 0), relu=True)
    pool = jnp.mean(h.astype(jnp.float32), axis=(1, 2, 3))
    return _finetune_head(pool, con0_w, con0_b, con1_w, con1_b, ft_w, ft_b)
